# Initial kernel scaffold; baseline (speedup 1.0000x reference)
import functools

import jax
import jax.numpy as jnp
from jax import lax
from jax.experimental import pallas as pl
from jax.experimental.pallas import tpu as pltpu
from jax.experimental.pallas import tpu_sc as plsc

N = 10000
E = 320000
NG = 64
NTILES = 16
K = 80
RPT = N // NTILES
BN_ROWS = 2500
SLOPE = 0.01


def _zero_rows(rows, w):
    def _z(r, carry):
        for kk in range(w // 16):
            rows[r, pl.ds(kk * 16, 16)] = jnp.zeros((16,), jnp.float32)
        return carry
    lax.fori_loop(0, K, _z, 0)


def _zero_acc_slice(rows, acc_sh, s):
    rb = s * RPT
    for i in range(RPT // K):
        pltpu.sync_copy(rows.at[pl.ds(0, K)], acc_sh.at[pl.ds(rb + i * K, K)])
    rem = RPT % K
    if rem:
        pltpu.sync_copy(rows.at[pl.ds(0, rem)],
                        acc_sh.at[pl.ds(rb + (RPT // K) * K, rem)])


def _make_agg(w, col_split):
    ec = E if col_split else E // 2
    et = ec // NTILES
    nch = et // K
    mesh = plsc.VectorSubcoreMesh(core_axis_name="c", subcore_axis_name="s")

    @functools.partial(
        pl.kernel,
        out_type=jax.ShapeDtypeStruct((2 * N, w), jnp.float32),
        mesh=mesh,
        scratch_types=[
            pltpu.VMEM((K,), jnp.int32),
            pltpu.VMEM((K,), jnp.int32),
            pltpu.VMEM((K, w), jnp.float32),
            pltpu.VMEM_SHARED((N, w), jnp.float32),
            pltpu.SemaphoreType.DMA,
        ],
    )
    def agg(t_hbm, src_hbm, dst_hbm, out_hbm, idx_s, idx_d, rows, acc_sh, sem):
        c = lax.axis_index("c")
        s = lax.axis_index("s")
        _zero_rows(rows, w)
        _zero_acc_slice(rows, acc_sh, s)
        plsc.subcore_barrier()

        if col_split:
            src_base = c * E + s * et
            dst_base = s * et
        else:
            src_base = c * (E // 2) + s * et
            dst_base = src_base

        def _chunk(j, carry):
            bs = src_base + j * K
            bd = dst_base + j * K
            pltpu.sync_copy(src_hbm.at[pl.ds(bs, K)], idx_s)
            pltpu.async_copy(t_hbm.at[idx_s], rows, sem).wait()
            pltpu.sync_copy(dst_hbm.at[pl.ds(bd, K)], idx_d)
            pltpu.sync_copy(rows, acc_sh.at[idx_d], add=True)
            return carry
        lax.fori_loop(0, nch, _chunk, 0)

        plsc.subcore_barrier()
        rb = s * RPT
        pltpu.sync_copy(acc_sh.at[pl.ds(rb, RPT)],
                        out_hbm.at[pl.ds(c * N + rb, RPT)])

    return agg


def _make_deg():
    w = 16
    et = (E // 2) // NTILES
    nch = et // K
    mesh = plsc.VectorSubcoreMesh(core_axis_name="c", subcore_axis_name="s")

    @functools.partial(
        pl.kernel,
        out_type=jax.ShapeDtypeStruct((2 * N, w), jnp.float32),
        mesh=mesh,
        scratch_types=[
            pltpu.VMEM((K,), jnp.int32),
            pltpu.VMEM((K, w), jnp.float32),
            pltpu.VMEM_SHARED((N, w), jnp.float32),
        ],
    )
    def deg(dst_hbm, out_hbm, idx_d, rows, acc_sh):
        c = lax.axis_index("c")
        s = lax.axis_index("s")
        _zero_rows(rows, w)
        _zero_acc_slice(rows, acc_sh, s)
        plsc.subcore_barrier()

        def _ones(r, carry):
            rows[r, pl.ds(0, 16)] = jnp.ones((16,), jnp.float32)
            return carry
        lax.fori_loop(0, K, _ones, 0)

        base = c * (E // 2) + s * et

        def _chunk(j, carry):
            pltpu.sync_copy(dst_hbm.at[pl.ds(base + j * K, K)], idx_d)
            pltpu.sync_copy(rows, acc_sh.at[idx_d], add=True)
            return carry
        lax.fori_loop(0, nch, _chunk, 0)

        plsc.subcore_barrier()
        rb = s * RPT
        pltpu.sync_copy(acc_sh.at[pl.ds(rb, RPT)],
                        out_hbm.at[pl.ds(c * N + rb, RPT)])

    return deg


_deg_k = _make_deg()
_agg128 = _make_agg(128, False)
_agg_cs128 = _make_agg(128, True)
_agg64 = _make_agg(64, False)
_agg32 = _make_agg(32, False)
_agg16 = _make_agg(16, False)


def _leaky(z):
    return jnp.where(z > 0, z, SLOPE * z)


def _pre_body(x_ref, deg2_ref, src_ref, dinv_ref, xs_ref, src2_ref):
    deg = deg2_ref[0, :, 0:1] + deg2_ref[1, :, 0:1] + 1.0
    dinv = lax.rsqrt(deg)
    dinv_ref[...] = dinv
    xs_ref[...] = x_ref[...] * dinv
    src2_ref[0:1, :] = src_ref[...]
    src2_ref[1:2, :] = src_ref[...] + N


def _pre(x, deg2, src):
    return pl.pallas_call(
        _pre_body,
        out_shape=(
            jax.ShapeDtypeStruct((N, 1), jnp.float32),
            jax.ShapeDtypeStruct((N, 128), jnp.float32),
            jax.ShapeDtypeStruct((2, E), jnp.int32),
        ),
    )(x, deg2.reshape(2, N, 16), src.reshape(1, E))


def _l1_stats_body(a2, xs, dinv, w1, b1, s_ref, s2_ref):
    i = pl.program_id(0)
    aggx = (a2[0] + a2[1] + xs[...]) * dinv[...]
    z = jnp.dot(aggx, w1[...], preferred_element_type=jnp.float32) + b1[...]
    a = _leaky(z)

    @pl.when(i == 0)
    def _():
        s_ref[...] = jnp.zeros_like(s_ref)
        s2_ref[...] = jnp.zeros_like(s2_ref)

    s_ref[...] += jnp.sum(a, axis=0, keepdims=True)
    s2_ref[...] += jnp.sum(a * a, axis=0, keepdims=True)


def _l1_apply_body(a2, xs, dinv, w1, b1, g1, be1, w2, s, s2, out):
    aggx = (a2[0] + a2[1] + xs[...]) * dinv[...]
    z = jnp.dot(aggx, w1[...], preferred_element_type=jnp.float32) + b1[...]
    a = _leaky(z)
    mu = s[...] / N
    var = s2[...] / N - mu * mu
    h = (a - mu) * lax.rsqrt(var + 1e-5) * g1[...] + be1[...]
    t = jnp.dot(h, w2[...], preferred_element_type=jnp.float32) * dinv[...]
    out[0] = t[:, 0:128]
    out[1] = t[:, 128:256]


def _layer1(acc2, xs, dinv, w1, b1, g1, be1, w2):
    grid = (N // BN_ROWS,)
    a2s = pl.BlockSpec((2, BN_ROWS, 128), lambda i: (0, i, 0))
    xss = pl.BlockSpec((BN_ROWS, 128), lambda i: (i, 0))
    dvs = pl.BlockSpec((BN_ROWS, 1), lambda i: (i, 0))
    w1s = pl.BlockSpec((128, 512), lambda i: (0, 0))
    vs512 = pl.BlockSpec((1, 512), lambda i: (0, 0))
    w2s = pl.BlockSpec((512, 256), lambda i: (0, 0))
    a2r = acc2.reshape(2, N, 128)
    s, s2 = pl.pallas_call(
        _l1_stats_body,
        grid=grid,
        in_specs=[a2s, xss, dvs, w1s, vs512],
        out_specs=(vs512, vs512),
        out_shape=(jax.ShapeDtypeStruct((1, 512), jnp.float32),
                   jax.ShapeDtypeStruct((1, 512), jnp.float32)),
    )(a2r, xs, dinv, w1, b1.reshape(1, 512))
    t2 = pl.pallas_call(
        _l1_apply_body,
        grid=grid,
        in_specs=[a2s, xss, dvs, w1s, vs512, vs512, vs512, w2s, vs512, vs512],
        out_specs=pl.BlockSpec((2, BN_ROWS, 128), lambda i: (0, i, 0)),
        out_shape=jax.ShapeDtypeStruct((2, N, 128), jnp.float32),
    )(a2r, xs, dinv, w1, b1.reshape(1, 512), g1.reshape(1, 512),
      be1.reshape(1, 512), w2, s, s2)
    return t2.reshape(2 * N, 128)


def _l2_stats_body(a2, t2, dinv, b2, s_ref, s2_ref):
    i = pl.program_id(0)
    z = jnp.concatenate([a2[0] + t2[0], a2[1] + t2[1]], axis=1)
    z = z * dinv[...] + b2[...]
    a = _leaky(z)

    @pl.when(i == 0)
    def _():
        s_ref[...] = jnp.zeros_like(s_ref)
        s2_ref[...] = jnp.zeros_like(s2_ref)

    s_ref[...] += jnp.sum(a, axis=0, keepdims=True)
    s2_ref[...] += jnp.sum(a * a, axis=0, keepdims=True)


def _l2_apply_body(a2, t2, dinv, b2, g2, be2, w3, s, s2, out):
    z = jnp.concatenate([a2[0] + t2[0], a2[1] + t2[1]], axis=1)
    z = z * dinv[...] + b2[...]
    a = _leaky(z)
    mu = s[...] / N
    var = s2[...] / N - mu * mu
    h = (a - mu) * lax.rsqrt(var + 1e-5) * g2[...] + be2[...]
    out[...] = jnp.dot(h, w3[...], preferred_element_type=jnp.float32) * dinv[...]


def _layer2(acc2, t2, dinv, b2, g2, be2, w3):
    grid = (N // BN_ROWS,)
    hs = pl.BlockSpec((2, BN_ROWS, 128), lambda i: (0, i, 0))
    dvs = pl.BlockSpec((BN_ROWS, 1), lambda i: (i, 0))
    vs256 = pl.BlockSpec((1, 256), lambda i: (0, 0))
    w3s = pl.BlockSpec((256, 128), lambda i: (0, 0))
    a2r = acc2.reshape(2, N, 128)
    t2r = t2.reshape(2, N, 128)
    s, s2 = pl.pallas_call(
        _l2_stats_body,
        grid=grid,
        in_specs=[hs, hs, dvs, vs256],
        out_specs=(vs256, vs256),
        out_shape=(jax.ShapeDtypeStruct((1, 256), jnp.float32),
                   jax.ShapeDtypeStruct((1, 256), jnp.float32)),
    )(a2r, t2r, dinv, b2.reshape(1, 256))
    t3 = pl.pallas_call(
        _l2_apply_body,
        grid=grid,
        in_specs=[hs, hs, dvs, vs256, vs256, vs256, w3s, vs256, vs256],
        out_specs=pl.BlockSpec((BN_ROWS, 128), lambda i: (i, 0)),
        out_shape=jax.ShapeDtypeStruct((N, 128), jnp.float32),
    )(a2r, t2r, dinv, b2.reshape(1, 256), g2.reshape(1, 256),
      be2.reshape(1, 256), w3, s, s2)
    return t3


def _mid_body(a2, tin, dinv, b, g, be, wn, out):
    z = (a2[0] + a2[1] + tin[...]) * dinv[...] + b[...]
    a = _leaky(z)
    mu = jnp.mean(a, axis=0, keepdims=True)
    var = jnp.mean((a - mu) * (a - mu), axis=0, keepdims=True)
    h = (a - mu) * lax.rsqrt(var + 1e-5) * g[...] + be[...]
    out[...] = jnp.dot(h, wn[...], preferred_element_type=jnp.float32) * dinv[...]


def _mid_layer(acc2, tin, dinv, b, g, be, wn, d, dn):
    return pl.pallas_call(
        _mid_body,
        out_shape=jax.ShapeDtypeStruct((N, dn), jnp.float32),
    )(acc2.reshape(2, N, d), tin, dinv, b.reshape(1, d), g.reshape(1, d),
      be.reshape(1, d), wn)


def _final_body(a2, tin, dinv, b6, batch, fc1w, fc1b, fc2w, fc2b, out):
    z = (a2[0] + a2[1] + tin[...]) * dinv[...] + b6[...]
    h6 = _leaky(z)
    gid = jax.lax.broadcasted_iota(jnp.int32, (N, NG), 1)
    m = (batch[...] == gid).astype(jnp.float32)
    sums = lax.dot_general(m, h6, (((0,), (0,)), ((), ())),
                           preferred_element_type=jnp.float32)
    cnt = jnp.sum(m, axis=0)[:, None]
    pooled = sums / jnp.maximum(cnt, 1.0)
    hf = _leaky(jnp.dot(pooled, fc1w[...],
                        preferred_element_type=jnp.float32) + fc1b[...])
    out[...] = jnp.dot(hf, fc2w[...],
                       preferred_element_type=jnp.float32) + fc2b[...]


def _final(acc2, tin, dinv, b6, batch, fc1w, fc1b, fc2w, fc2b):
    return pl.pallas_call(
        _final_body,
        out_shape=jax.ShapeDtypeStruct((NG, 1), jnp.float32),
    )(acc2.reshape(2, N, 16), tin, dinv, b6.reshape(1, 16),
      batch.reshape(N, 1), fc1w, fc1b.reshape(1, 8), fc2w, fc2b.reshape(1, 1))


@jax.jit
def _run(x, edge_index, batch,
         W1, b1, g1, be1, W2, b2, g2, be2, W3, b3, g3, be3,
         W4, b4, g4, be4, W5, b5, g5, be5, W6, b6,
         fc1_W, fc1_b, fc2_W, fc2_b):
    src = edge_index[0]
    dst = edge_index[1]
    deg2 = _deg_k(dst)
    dinv, xs, src2 = _pre(x, deg2, src)
    src2 = src2.reshape(2 * E)

    acc1 = _agg128(xs, src, dst)
    t2 = _layer1(acc1, xs, dinv, W1, b1, g1, be1, W2)

    acc2 = _agg_cs128(t2, src2, dst)
    t3 = _layer2(acc2, t2, dinv, b2, g2, be2, W3)

    acc3 = _agg128(t3, src, dst)
    t4 = _mid_layer(acc3, t3, dinv, b3, g3, be3, W4, 128, 64)

    acc4 = _agg64(t4, src, dst)
    t5 = _mid_layer(acc4, t4, dinv, b4, g4, be4, W5, 64, 32)

    acc5 = _agg32(t5, src, dst)
    t6 = _mid_layer(acc5, t5, dinv, b5, g5, be5, W6, 32, 16)

    acc6 = _agg16(t6, src, dst)
    return _final(acc6, t6, dinv, b6, batch, fc1_W, fc1_b, fc2_W, fc2_b)


def kernel(x, edge_index, batch,
           W1, b1, g1, be1, W2, b2, g2, be2, W3, b3, g3, be3,
           W4, b4, g4, be4, W5, b5, g5, be5, W6, b6,
           fc1_W, fc1_b, fc2_W, fc2_b):
    return _run(x, edge_index, batch,
                W1, b1, g1, be1, W2, b2, g2, be2, W3, b3, g3, be3,
                W4, b4, g4, be4, W5, b5, g5, be5, W6, b6,
                fc1_W, fc1_b, fc2_W, fc2_b)


# speedup vs baseline: 9.6632x; 9.6632x over previous
import functools

import jax
import jax.numpy as jnp
from jax import lax
from jax.experimental import pallas as pl
from jax.experimental.pallas import tpu as pltpu
from jax.experimental.pallas import tpu_sc as plsc

N = 10000
E = 320000
NG = 64
NTILES = 16
K = 80
RPT = N // NTILES
BN_ROWS = 2000
SLOPE = 0.01


def _zero_rows(rows, w):
    def _z(r, carry):
        for kk in range(w // 16):
            rows[r, pl.ds(kk * 16, 16)] = jnp.zeros((16,), jnp.float32)
        return carry
    lax.fori_loop(0, K, _z, 0)


NCHROWS = N // K
ROUNDS = (NCHROWS + NTILES - 1) // NTILES


def _zero_acc_slice(rows, acc_sh, s):
    for r in range(ROUNDS):
        ch = s + r * NTILES

        @pl.when(ch < NCHROWS)
        def _():
            pltpu.sync_copy(rows.at[pl.ds(0, K)], acc_sh.at[pl.ds(ch * K, K)])


def _writeout(acc_sh, out_hbm, c, s):
    for r in range(ROUNDS):
        ch = s + r * NTILES

        @pl.when(ch < NCHROWS)
        def _():
            pltpu.sync_copy(acc_sh.at[pl.ds(ch * K, K)],
                            out_hbm.at[pl.ds(c * N + ch * K, K)])


def _make_agg(w, col_split):
    ec = E if col_split else E // 2
    et = ec // NTILES
    nch = et // K
    mesh = plsc.VectorSubcoreMesh(core_axis_name="c", subcore_axis_name="s", num_cores=2, num_subcores=16)

    @functools.partial(
        pl.kernel,
        out_type=jax.ShapeDtypeStruct((2 * N, w), jnp.float32),
        mesh=mesh,
        scratch_types=[
            pltpu.VMEM((K,), jnp.int32),
            pltpu.VMEM((K,), jnp.int32),
            pltpu.VMEM((K, w), jnp.float32),
            pltpu.VMEM_SHARED((N, w), jnp.float32),
            pltpu.SemaphoreType.DMA,
        ],
    )
    def agg(t_hbm, src_hbm, dst_hbm, out_hbm, idx_s, idx_d, rows, acc_sh, sem):
        c = lax.axis_index("c")
        s = lax.axis_index("s")
        _zero_rows(rows, w)
        _zero_acc_slice(rows, acc_sh, s)
        plsc.subcore_barrier()

        if col_split:
            src_base = c * E + s * et
            dst_base = s * et
        else:
            src_base = c * (E // 2) + s * et
            dst_base = src_base

        def _chunk(j, carry):
            bs = src_base + j * K
            bd = dst_base + j * K
            pltpu.sync_copy(src_hbm.at[pl.ds(bs, K)], idx_s)
            pltpu.async_copy(t_hbm.at[idx_s], rows, sem).wait()
            pltpu.sync_copy(dst_hbm.at[pl.ds(bd, K)], idx_d)
            pltpu.sync_copy(rows, acc_sh.at[idx_d], add=True)
            return carry
        lax.fori_loop(0, nch, _chunk, 0)

        plsc.subcore_barrier()
        _writeout(acc_sh, out_hbm, c, s)

    return agg


def _make_deg():
    w = 128
    et = (E // 2) // NTILES
    nch = et // K
    mesh = plsc.VectorSubcoreMesh(core_axis_name="c", subcore_axis_name="s", num_cores=2, num_subcores=16)

    @functools.partial(
        pl.kernel,
        out_type=jax.ShapeDtypeStruct((2 * N, 128), jnp.float32),
        mesh=mesh,
        scratch_types=[
            pltpu.VMEM((K,), jnp.int32),
            pltpu.VMEM((K, 128), jnp.float32),
            pltpu.VMEM_SHARED((N, 128), jnp.float32),
        ],
    )
    def deg(dst_hbm, out_hbm, idx_d, rows, acc_sh):
        c = lax.axis_index("c")
        s = lax.axis_index("s")
        _zero_rows(rows, w)
        _zero_acc_slice(rows, acc_sh, s)
        plsc.subcore_barrier()

        def _ones(r, carry):
            for kk in range(8):
                rows[r, pl.ds(kk * 16, 16)] = jnp.ones((16,), jnp.float32)
            return carry
        lax.fori_loop(0, K, _ones, 0)

        base = c * (E // 2) + s * et

        def _chunk(j, carry):
            pltpu.sync_copy(dst_hbm.at[pl.ds(base + j * K, K)], idx_d)
            pltpu.sync_copy(rows, acc_sh.at[idx_d], add=True)
            return carry
        lax.fori_loop(0, nch, _chunk, 0)

        plsc.subcore_barrier()
        _writeout(acc_sh, out_hbm, c, s)

    return deg


@functools.lru_cache(maxsize=None)
def _sc_kernels():
    return {
        "deg": _make_deg(),
        "a128": _make_agg(128, False),
        "cs128": _make_agg(128, True),
    }


def _leaky(z):
    return jnp.where(z > 0, z, SLOPE * z)


def _pre_body(x_ref, deg2_ref, src_ref, dinv_ref, xs_ref, src2_ref):
    deg = deg2_ref[0, :, 0:1] + deg2_ref[1, :, 0:1] + 1.0
    dinv = lax.rsqrt(deg)
    dinv_ref[...] = dinv
    xs_ref[...] = x_ref[...] * dinv
    src2_ref[0:1, :] = src_ref[...]
    src2_ref[1:2, :] = src_ref[...] + N


def _pre(x, deg2, src):
    return pl.pallas_call(
        _pre_body,
        out_shape=(
            jax.ShapeDtypeStruct((N, 1), jnp.float32),
            jax.ShapeDtypeStruct((N, 128), jnp.float32),
            jax.ShapeDtypeStruct((2, E), jnp.int32),
        ),
    )(x, deg2.reshape(2, N, 128), src.reshape(1, E))


def _l1_stats_body(a2, xs, dinv, w1, b1, s_ref, s2_ref):
    i = pl.program_id(0)
    aggx = (a2[0] + a2[1] + xs[...]) * dinv[...]
    z = jnp.dot(aggx, w1[...], preferred_element_type=jnp.float32) + b1[...]
    a = _leaky(z)

    @pl.when(i == 0)
    def _():
        s_ref[...] = jnp.zeros_like(s_ref)
        s2_ref[...] = jnp.zeros_like(s2_ref)

    s_ref[...] += jnp.sum(a, axis=0, keepdims=True)
    s2_ref[...] += jnp.sum(a * a, axis=0, keepdims=True)


def _l1_apply_body(a2, xs, dinv, w1, b1, g1, be1, w2, s, s2, out):
    aggx = (a2[0] + a2[1] + xs[...]) * dinv[...]
    z = jnp.dot(aggx, w1[...], preferred_element_type=jnp.float32) + b1[...]
    a = _leaky(z)
    mu = s[...] / N
    var = s2[...] / N - mu * mu
    h = (a - mu) * lax.rsqrt(var + 1e-5) * g1[...] + be1[...]
    t = jnp.dot(h, w2[...], preferred_element_type=jnp.float32) * dinv[...]
    out[0] = t[:, 0:128]
    out[1] = t[:, 128:256]


def _layer1(acc2, xs, dinv, w1, b1, g1, be1, w2):
    grid = (N // BN_ROWS,)
    a2s = pl.BlockSpec((2, BN_ROWS, 128), lambda i: (0, i, 0))
    xss = pl.BlockSpec((BN_ROWS, 128), lambda i: (i, 0))
    dvs = pl.BlockSpec((BN_ROWS, 1), lambda i: (i, 0))
    w1s = pl.BlockSpec((128, 512), lambda i: (0, 0))
    vs512 = pl.BlockSpec((1, 512), lambda i: (0, 0))
    w2s = pl.BlockSpec((512, 256), lambda i: (0, 0))
    a2r = acc2.reshape(2, N, 128)
    s, s2 = pl.pallas_call(
        _l1_stats_body,
        grid=grid,
        in_specs=[a2s, xss, dvs, w1s, vs512],
        out_specs=(vs512, vs512),
        out_shape=(jax.ShapeDtypeStruct((1, 512), jnp.float32),
                   jax.ShapeDtypeStruct((1, 512), jnp.float32)),
    )(a2r, xs, dinv, w1, b1.reshape(1, 512))
    t2 = pl.pallas_call(
        _l1_apply_body,
        grid=grid,
        in_specs=[a2s, xss, dvs, w1s, vs512, vs512, vs512, w2s, vs512, vs512],
        out_specs=pl.BlockSpec((2, BN_ROWS, 128), lambda i: (0, i, 0)),
        out_shape=jax.ShapeDtypeStruct((2, N, 128), jnp.float32),
    )(a2r, xs, dinv, w1, b1.reshape(1, 512), g1.reshape(1, 512),
      be1.reshape(1, 512), w2, s, s2)
    return t2.reshape(2 * N, 128)


def _l2_stats_body(a2, t2, dinv, b2, s_ref, s2_ref):
    i = pl.program_id(0)
    z = jnp.concatenate([a2[0] + t2[0], a2[1] + t2[1]], axis=1)
    z = z * dinv[...] + b2[...]
    a = _leaky(z)

    @pl.when(i == 0)
    def _():
        s_ref[...] = jnp.zeros_like(s_ref)
        s2_ref[...] = jnp.zeros_like(s2_ref)

    s_ref[...] += jnp.sum(a, axis=0, keepdims=True)
    s2_ref[...] += jnp.sum(a * a, axis=0, keepdims=True)


def _l2_apply_body(a2, t2, dinv, b2, g2, be2, w3, s, s2, out):
    z = jnp.concatenate([a2[0] + t2[0], a2[1] + t2[1]], axis=1)
    z = z * dinv[...] + b2[...]
    a = _leaky(z)
    mu = s[...] / N
    var = s2[...] / N - mu * mu
    h = (a - mu) * lax.rsqrt(var + 1e-5) * g2[...] + be2[...]
    out[...] = jnp.dot(h, w3[...], preferred_element_type=jnp.float32) * dinv[...]


def _layer2(acc2, t2, dinv, b2, g2, be2, w3):
    grid = (N // BN_ROWS,)
    hs = pl.BlockSpec((2, BN_ROWS, 128), lambda i: (0, i, 0))
    dvs = pl.BlockSpec((BN_ROWS, 1), lambda i: (i, 0))
    vs256 = pl.BlockSpec((1, 256), lambda i: (0, 0))
    w3s = pl.BlockSpec((256, 128), lambda i: (0, 0))
    a2r = acc2.reshape(2, N, 128)
    t2r = t2.reshape(2, N, 128)
    s, s2 = pl.pallas_call(
        _l2_stats_body,
        grid=grid,
        in_specs=[hs, hs, dvs, vs256],
        out_specs=(vs256, vs256),
        out_shape=(jax.ShapeDtypeStruct((1, 256), jnp.float32),
                   jax.ShapeDtypeStruct((1, 256), jnp.float32)),
    )(a2r, t2r, dinv, b2.reshape(1, 256))
    t3 = pl.pallas_call(
        _l2_apply_body,
        grid=grid,
        in_specs=[hs, hs, dvs, vs256, vs256, vs256, w3s, vs256, vs256],
        out_specs=pl.BlockSpec((BN_ROWS, 128), lambda i: (i, 0)),
        out_shape=jax.ShapeDtypeStruct((N, 128), jnp.float32),
    )(a2r, t2r, dinv, b2.reshape(1, 256), g2.reshape(1, 256),
      be2.reshape(1, 256), w3, s, s2)
    return t3


def _make_mid_body(d, dn):
    def _mid_body(a2, tin, dinv, b, g, be, wn, out):
        z = (a2[0][:, 0:d] + a2[1][:, 0:d] + tin[:, 0:d]) * dinv[...] + b[...]
        a = _leaky(z)
        mu = jnp.mean(a, axis=0, keepdims=True)
        var = jnp.mean((a - mu) * (a - mu), axis=0, keepdims=True)
        h = (a - mu) * lax.rsqrt(var + 1e-5) * g[...] + be[...]
        t = jnp.dot(h, wn[...], preferred_element_type=jnp.float32) * dinv[...]
        out[:, 0:dn] = t
        out[:, dn:128] = jnp.zeros_like(out[:, dn:128])
    return _mid_body


def _mid_layer(acc2, tin, dinv, b, g, be, wn, d, dn):
    return pl.pallas_call(
        _make_mid_body(d, dn),
        out_shape=jax.ShapeDtypeStruct((N, 128), jnp.float32),
    )(acc2.reshape(2, N, 128), tin, dinv, b.reshape(1, d), g.reshape(1, d),
      be.reshape(1, d), wn)


def _final_body(a2, tin, dinv, b6, batch, fc1w, fc1b, fc2w, fc2b, out):
    z = (a2[0][:, 0:16] + a2[1][:, 0:16] + tin[:, 0:16]) * dinv[...] + b6[...]
    h6 = _leaky(z)
    gid = jax.lax.broadcasted_iota(jnp.int32, (N, NG), 1)
    m = (batch[...] == gid).astype(jnp.float32)
    sums = lax.dot_general(m, h6, (((0,), (0,)), ((), ())),
                           preferred_element_type=jnp.float32)
    cnt = jnp.sum(m, axis=0)[:, None]
    pooled = sums / jnp.maximum(cnt, 1.0)
    hf = _leaky(jnp.dot(pooled, fc1w[...],
                        preferred_element_type=jnp.float32) + fc1b[...])
    out[...] = jnp.dot(hf, fc2w[...],
                       preferred_element_type=jnp.float32) + fc2b[...]


def _final(acc2, tin, dinv, b6, batch, fc1w, fc1b, fc2w, fc2b):
    return pl.pallas_call(
        _final_body,
        out_shape=jax.ShapeDtypeStruct((NG, 1), jnp.float32),
    )(acc2.reshape(2, N, 128), tin, dinv, b6.reshape(1, 16),
      batch.reshape(N, 1), fc1w, fc1b.reshape(1, 8), fc2w, fc2b.reshape(1, 1))


@jax.jit
def _run(x, edge_index, batch,
         W1, b1, g1, be1, W2, b2, g2, be2, W3, b3, g3, be3,
         W4, b4, g4, be4, W5, b5, g5, be5, W6, b6,
         fc1_W, fc1_b, fc2_W, fc2_b):
    sc = _sc_kernels()
    src = edge_index[0]
    dst = edge_index[1]
    deg2 = sc["deg"](dst)
    dinv, xs, src2 = _pre(x, deg2, src)
    src2 = src2.reshape(2 * E)

    acc1 = sc["a128"](xs, src, dst)
    t2 = _layer1(acc1, xs, dinv, W1, b1, g1, be1, W2)

    acc2 = sc["cs128"](t2, src2, dst)
    t3 = _layer2(acc2, t2, dinv, b2, g2, be2, W3)

    acc3 = sc["a128"](t3, src, dst)
    t4 = _mid_layer(acc3, t3, dinv, b3, g3, be3, W4, 128, 64)

    acc4 = sc["a128"](t4, src, dst)
    t5 = _mid_layer(acc4, t4, dinv, b4, g4, be4, W5, 64, 32)

    acc5 = sc["a128"](t5, src, dst)
    t6 = _mid_layer(acc5, t5, dinv, b5, g5, be5, W6, 32, 16)

    acc6 = sc["a128"](t6, src, dst)
    return _final(acc6, t6, dinv, b6, batch, fc1_W, fc1_b, fc2_W, fc2_b)


def kernel(x, edge_index, batch,
           W1, b1, g1, be1, W2, b2, g2, be2, W3, b3, g3, be3,
           W4, b4, g4, be4, W5, b5, g5, be5, W6, b6,
           fc1_W, fc1_b, fc2_W, fc2_b):
    return _run(x, edge_index, batch,
                W1, b1, g1, be1, W2, b2, g2, be2, W3, b3, g3, be3,
                W4, b4, g4, be4, W5, b5, g5, be5, W6, b6,
                fc1_W, fc1_b, fc2_W, fc2_b)


# speedup vs baseline: 17.6247x; 1.8239x over previous
import functools

import jax
import jax.numpy as jnp
from jax import lax
from jax.experimental import pallas as pl
from jax.experimental.pallas import tpu as pltpu
from jax.experimental.pallas import tpu_sc as plsc

N = 10000
E = 320000
NG = 64
NTILES = 16
K = 80
RPT = N // NTILES
BN_ROWS = 2000
SLOPE = 0.01


def _zero_rows(rows, w):
    def _z(r, carry):
        for kk in range(w // 16):
            rows[r, pl.ds(kk * 16, 16)] = jnp.zeros((16,), jnp.float32)
        return carry
    lax.fori_loop(0, K, _z, 0)


NCHROWS = N // K
ROUNDS = (NCHROWS + NTILES - 1) // NTILES


def _zero_acc_slice(rows, acc_sh, s):
    for r in range(ROUNDS):
        ch = s + r * NTILES

        @pl.when(ch < NCHROWS)
        def _():
            pltpu.sync_copy(rows.at[pl.ds(0, K)], acc_sh.at[pl.ds(ch * K, K)])


def _writeout(acc_sh, out_hbm, c, s):
    for r in range(ROUNDS):
        ch = s + r * NTILES

        @pl.when(ch < NCHROWS)
        def _():
            pltpu.sync_copy(acc_sh.at[pl.ds(ch * K, K)],
                            out_hbm.at[pl.ds(c * N + ch * K, K)])


def _make_agg(w, col_split):
    ec = E if col_split else E // 2
    et = ec // NTILES
    kc = 128
    npair = et // (2 * kc)
    tail = et - npair * 2 * kc
    mesh = plsc.VectorSubcoreMesh(core_axis_name="c", subcore_axis_name="s", num_cores=2, num_subcores=16)

    @functools.partial(
        pl.kernel,
        out_type=jax.ShapeDtypeStruct((2 * N, w), jnp.float32),
        mesh=mesh,
        scratch_types=[
            pltpu.VMEM((kc,), jnp.int32),
            pltpu.VMEM((kc,), jnp.int32),
            pltpu.VMEM((kc,), jnp.int32),
            pltpu.VMEM((kc,), jnp.int32),
            pltpu.VMEM((tail,), jnp.int32),
            pltpu.VMEM((tail,), jnp.int32),
            pltpu.VMEM((K, w), jnp.float32),
            pltpu.VMEM((kc, w), jnp.float32),
            pltpu.VMEM((kc, w), jnp.float32),
            pltpu.VMEM_SHARED((N, w), jnp.float32),
            pltpu.SemaphoreType.DMA,
            pltpu.SemaphoreType.DMA,
        ],
    )
    def agg(t_hbm, src_hbm, dst_hbm, out_hbm, isa, isb, ida, idb, ist, idt,
            zrows, rows_a, rows_b, acc_sh, sema, semb):
        c = lax.axis_index("c")
        s = lax.axis_index("s")
        _zero_rows(zrows, w)
        _zero_acc_slice(zrows, acc_sh, s)
        plsc.subcore_barrier()

        if col_split:
            src_base = c * E + s * et
            dst_base = s * et
        else:
            src_base = c * (E // 2) + s * et
            dst_base = src_base

        def _pair(jj, carry):
            b0 = jj * 2 * kc
            b1 = b0 + kc
            pltpu.sync_copy(src_hbm.at[pl.ds(src_base + b0, kc)], isa)
            ga = pltpu.async_copy(t_hbm.at[isa], rows_a, sema)
            pltpu.sync_copy(src_hbm.at[pl.ds(src_base + b1, kc)], isb)
            gb = pltpu.async_copy(t_hbm.at[isb], rows_b, semb)
            pltpu.sync_copy(dst_hbm.at[pl.ds(dst_base + b0, kc)], ida)
            pltpu.sync_copy(dst_hbm.at[pl.ds(dst_base + b1, kc)], idb)
            ga.wait()
            pltpu.sync_copy(rows_a, acc_sh.at[ida], add=True)
            gb.wait()
            pltpu.sync_copy(rows_b, acc_sh.at[idb], add=True)
            return carry
        lax.fori_loop(0, npair, _pair, 0)

        if tail:
            bt = npair * 2 * kc
            pltpu.sync_copy(src_hbm.at[pl.ds(src_base + bt, tail)], ist)
            gt = pltpu.async_copy(t_hbm.at[ist], rows_a.at[pl.ds(0, tail)],
                                  sema)
            pltpu.sync_copy(dst_hbm.at[pl.ds(dst_base + bt, tail)], idt)
            gt.wait()
            pltpu.sync_copy(rows_a.at[pl.ds(0, tail)], acc_sh.at[idt],
                            add=True)

        plsc.subcore_barrier()
        _writeout(acc_sh, out_hbm, c, s)

    return agg


def _make_deg():
    w = 128
    et = (E // 2) // NTILES
    nch = et // K
    mesh = plsc.VectorSubcoreMesh(core_axis_name="c", subcore_axis_name="s", num_cores=2, num_subcores=16)

    @functools.partial(
        pl.kernel,
        out_type=jax.ShapeDtypeStruct((2 * N, 128), jnp.float32),
        mesh=mesh,
        scratch_types=[
            pltpu.VMEM((K,), jnp.int32),
            pltpu.VMEM((K,), jnp.int32),
            pltpu.VMEM((K, 128), jnp.float32),
            pltpu.VMEM_SHARED((N, 128), jnp.float32),
            pltpu.SemaphoreType.DMA,
            pltpu.SemaphoreType.DMA,
        ],
    )
    def deg(dst_hbm, out_hbm, idx_d, idx_d2, rows, acc_sh, sem_a, sem_b):
        c = lax.axis_index("c")
        s = lax.axis_index("s")
        _zero_rows(rows, w)
        _zero_acc_slice(rows, acc_sh, s)
        plsc.subcore_barrier()

        def _ones(r, carry):
            for kk in range(8):
                rows[r, pl.ds(kk * 16, 16)] = jnp.ones((16,), jnp.float32)
            return carry
        lax.fori_loop(0, K, _ones, 0)

        base = c * (E // 2) + s * et

        def _chunk(j, carry):
            ia = pltpu.async_copy(dst_hbm.at[pl.ds(base + j * 2 * K, K)],
                                  idx_d, sem_a)
            ib = pltpu.async_copy(dst_hbm.at[pl.ds(base + j * 2 * K + K, K)],
                                  idx_d2, sem_b)
            ia.wait()
            pltpu.sync_copy(rows, acc_sh.at[idx_d], add=True)
            ib.wait()
            pltpu.sync_copy(rows, acc_sh.at[idx_d2], add=True)
            return carry
        lax.fori_loop(0, nch // 2, _chunk, 0)

        if nch % 2:
            pltpu.sync_copy(dst_hbm.at[pl.ds(base + (nch - 1) * K, K)], idx_d)
            pltpu.sync_copy(rows, acc_sh.at[idx_d], add=True)

        plsc.subcore_barrier()
        _writeout(acc_sh, out_hbm, c, s)

    return deg


@functools.lru_cache(maxsize=None)
def _sc_kernels():
    return {
        "deg": _make_deg(),
        "a128": _make_agg(128, False),
        "cs128": _make_agg(128, True),
    }


def _leaky(z):
    return jnp.where(z > 0, z, SLOPE * z)


def _pre_body(x_ref, deg2_ref, src_ref, dinv_ref, xs_ref, src2_ref):
    deg = deg2_ref[0, :, 0:1] + deg2_ref[1, :, 0:1] + 1.0
    dinv = lax.rsqrt(deg)
    dinv_ref[...] = dinv
    xs_ref[...] = x_ref[...] * dinv
    src2_ref[0:1, :] = src_ref[...]
    src2_ref[1:2, :] = src_ref[...] + N


def _pre(x, deg2, src):
    return pl.pallas_call(
        _pre_body,
        out_shape=(
            jax.ShapeDtypeStruct((N, 1), jnp.float32),
            jax.ShapeDtypeStruct((N, 128), jnp.float32),
            jax.ShapeDtypeStruct((2, E), jnp.int32),
        ),
    )(x, deg2.reshape(2, N, 128), src.reshape(1, E))


def _l1_stats_body(a2, xs, dinv, w1, b1, s_ref, s2_ref):
    i = pl.program_id(0)
    aggx = (a2[0] + a2[1] + xs[...]) * dinv[...]
    z = jnp.dot(aggx, w1[...], preferred_element_type=jnp.float32) + b1[...]
    a = _leaky(z)

    @pl.when(i == 0)
    def _():
        s_ref[...] = jnp.zeros_like(s_ref)
        s2_ref[...] = jnp.zeros_like(s2_ref)

    s_ref[...] += jnp.sum(a, axis=0, keepdims=True)
    s2_ref[...] += jnp.sum(a * a, axis=0, keepdims=True)


def _l1_apply_body(a2, xs, dinv, w1, b1, g1, be1, w2, s, s2, out):
    aggx = (a2[0] + a2[1] + xs[...]) * dinv[...]
    z = jnp.dot(aggx, w1[...], preferred_element_type=jnp.float32) + b1[...]
    a = _leaky(z)
    mu = s[...] / N
    var = s2[...] / N - mu * mu
    h = (a - mu) * lax.rsqrt(var + 1e-5) * g1[...] + be1[...]
    t = jnp.dot(h, w2[...], preferred_element_type=jnp.float32) * dinv[...]
    out[0] = t[:, 0:128]
    out[1] = t[:, 128:256]


def _layer1(acc2, xs, dinv, w1, b1, g1, be1, w2):
    grid = (N // BN_ROWS,)
    a2s = pl.BlockSpec((2, BN_ROWS, 128), lambda i: (0, i, 0))
    xss = pl.BlockSpec((BN_ROWS, 128), lambda i: (i, 0))
    dvs = pl.BlockSpec((BN_ROWS, 1), lambda i: (i, 0))
    w1s = pl.BlockSpec((128, 512), lambda i: (0, 0))
    vs512 = pl.BlockSpec((1, 512), lambda i: (0, 0))
    w2s = pl.BlockSpec((512, 256), lambda i: (0, 0))
    a2r = acc2.reshape(2, N, 128)
    s, s2 = pl.pallas_call(
        _l1_stats_body,
        grid=grid,
        in_specs=[a2s, xss, dvs, w1s, vs512],
        out_specs=(vs512, vs512),
        out_shape=(jax.ShapeDtypeStruct((1, 512), jnp.float32),
                   jax.ShapeDtypeStruct((1, 512), jnp.float32)),
    )(a2r, xs, dinv, w1, b1.reshape(1, 512))
    t2 = pl.pallas_call(
        _l1_apply_body,
        grid=grid,
        in_specs=[a2s, xss, dvs, w1s, vs512, vs512, vs512, w2s, vs512, vs512],
        out_specs=pl.BlockSpec((2, BN_ROWS, 128), lambda i: (0, i, 0)),
        out_shape=jax.ShapeDtypeStruct((2, N, 128), jnp.float32),
    )(a2r, xs, dinv, w1, b1.reshape(1, 512), g1.reshape(1, 512),
      be1.reshape(1, 512), w2, s, s2)
    return t2.reshape(2 * N, 128)


def _l2_stats_body(a2, t2, dinv, b2, s_ref, s2_ref):
    i = pl.program_id(0)
    z = jnp.concatenate([a2[0] + t2[0], a2[1] + t2[1]], axis=1)
    z = z * dinv[...] + b2[...]
    a = _leaky(z)

    @pl.when(i == 0)
    def _():
        s_ref[...] = jnp.zeros_like(s_ref)
        s2_ref[...] = jnp.zeros_like(s2_ref)

    s_ref[...] += jnp.sum(a, axis=0, keepdims=True)
    s2_ref[...] += jnp.sum(a * a, axis=0, keepdims=True)


def _l2_apply_body(a2, t2, dinv, b2, g2, be2, w3, s, s2, out):
    z = jnp.concatenate([a2[0] + t2[0], a2[1] + t2[1]], axis=1)
    z = z * dinv[...] + b2[...]
    a = _leaky(z)
    mu = s[...] / N
    var = s2[...] / N - mu * mu
    h = (a - mu) * lax.rsqrt(var + 1e-5) * g2[...] + be2[...]
    out[...] = jnp.dot(h, w3[...], preferred_element_type=jnp.float32) * dinv[...]


def _layer2(acc2, t2, dinv, b2, g2, be2, w3):
    grid = (N // BN_ROWS,)
    hs = pl.BlockSpec((2, BN_ROWS, 128), lambda i: (0, i, 0))
    dvs = pl.BlockSpec((BN_ROWS, 1), lambda i: (i, 0))
    vs256 = pl.BlockSpec((1, 256), lambda i: (0, 0))
    w3s = pl.BlockSpec((256, 128), lambda i: (0, 0))
    a2r = acc2.reshape(2, N, 128)
    t2r = t2.reshape(2, N, 128)
    s, s2 = pl.pallas_call(
        _l2_stats_body,
        grid=grid,
        in_specs=[hs, hs, dvs, vs256],
        out_specs=(vs256, vs256),
        out_shape=(jax.ShapeDtypeStruct((1, 256), jnp.float32),
                   jax.ShapeDtypeStruct((1, 256), jnp.float32)),
    )(a2r, t2r, dinv, b2.reshape(1, 256))
    t3 = pl.pallas_call(
        _l2_apply_body,
        grid=grid,
        in_specs=[hs, hs, dvs, vs256, vs256, vs256, w3s, vs256, vs256],
        out_specs=pl.BlockSpec((BN_ROWS, 128), lambda i: (i, 0)),
        out_shape=jax.ShapeDtypeStruct((N, 128), jnp.float32),
    )(a2r, t2r, dinv, b2.reshape(1, 256), g2.reshape(1, 256),
      be2.reshape(1, 256), w3, s, s2)
    return t3


def _make_mid_body(d, dn):
    def _mid_body(a2, tin, dinv, b, g, be, wn, out):
        z = (a2[0][:, 0:d] + a2[1][:, 0:d] + tin[:, 0:d]) * dinv[...] + b[...]
        a = _leaky(z)
        mu = jnp.mean(a, axis=0, keepdims=True)
        var = jnp.mean((a - mu) * (a - mu), axis=0, keepdims=True)
        h = (a - mu) * lax.rsqrt(var + 1e-5) * g[...] + be[...]
        t = jnp.dot(h, wn[...], preferred_element_type=jnp.float32) * dinv[...]
        out[:, 0:dn] = t
        out[:, dn:128] = jnp.zeros_like(out[:, dn:128])
    return _mid_body


def _mid_layer(acc2, tin, dinv, b, g, be, wn, d, dn):
    return pl.pallas_call(
        _make_mid_body(d, dn),
        out_shape=jax.ShapeDtypeStruct((N, 128), jnp.float32),
    )(acc2.reshape(2, N, 128), tin, dinv, b.reshape(1, d), g.reshape(1, d),
      be.reshape(1, d), wn)


def _final_body(a2, tin, dinv, b6, batch, fc1w, fc1b, fc2w, fc2b, out):
    z = (a2[0][:, 0:16] + a2[1][:, 0:16] + tin[:, 0:16]) * dinv[...] + b6[...]
    h6 = _leaky(z)
    gid = jax.lax.broadcasted_iota(jnp.int32, (N, NG), 1)
    m = (batch[...] == gid).astype(jnp.float32)
    sums = lax.dot_general(m, h6, (((0,), (0,)), ((), ())),
                           preferred_element_type=jnp.float32)
    cnt = jnp.sum(m, axis=0)[:, None]
    pooled = sums / jnp.maximum(cnt, 1.0)
    hf = _leaky(jnp.dot(pooled, fc1w[...],
                        preferred_element_type=jnp.float32) + fc1b[...])
    out[...] = jnp.dot(hf, fc2w[...],
                       preferred_element_type=jnp.float32) + fc2b[...]


def _final(acc2, tin, dinv, b6, batch, fc1w, fc1b, fc2w, fc2b):
    return pl.pallas_call(
        _final_body,
        out_shape=jax.ShapeDtypeStruct((NG, 1), jnp.float32),
    )(acc2.reshape(2, N, 128), tin, dinv, b6.reshape(1, 16),
      batch.reshape(N, 1), fc1w, fc1b.reshape(1, 8), fc2w, fc2b.reshape(1, 1))


@jax.jit
def _run(x, edge_index, batch,
         W1, b1, g1, be1, W2, b2, g2, be2, W3, b3, g3, be3,
         W4, b4, g4, be4, W5, b5, g5, be5, W6, b6,
         fc1_W, fc1_b, fc2_W, fc2_b):
    sc = _sc_kernels()
    src = edge_index[0]
    dst = edge_index[1]
    deg2 = sc["deg"](dst)
    dinv, xs, src2 = _pre(x, deg2, src)
    src2 = src2.reshape(2 * E)

    acc1 = sc["a128"](xs, src, dst)
    t2 = _layer1(acc1, xs, dinv, W1, b1, g1, be1, W2)

    acc2 = sc["cs128"](t2, src2, dst)
    t3 = _layer2(acc2, t2, dinv, b2, g2, be2, W3)

    acc3 = sc["a128"](t3, src, dst)
    t4 = _mid_layer(acc3, t3, dinv, b3, g3, be3, W4, 128, 64)

    acc4 = sc["a128"](t4, src, dst)
    t5 = _mid_layer(acc4, t4, dinv, b4, g4, be4, W5, 64, 32)

    acc5 = sc["a128"](t5, src, dst)
    t6 = _mid_layer(acc5, t5, dinv, b5, g5, be5, W6, 32, 16)

    acc6 = sc["a128"](t6, src, dst)
    return _final(acc6, t6, dinv, b6, batch, fc1_W, fc1_b, fc2_W, fc2_b)


def kernel(x, edge_index, batch,
           W1, b1, g1, be1, W2, b2, g2, be2, W3, b3, g3, be3,
           W4, b4, g4, be4, W5, b5, g5, be5, W6, b6,
           fc1_W, fc1_b, fc2_W, fc2_b):
    return _run(x, edge_index, batch,
                W1, b1, g1, be1, W2, b2, g2, be2, W3, b3, g3, be3,
                W4, b4, g4, be4, W5, b5, g5, be5, W6, b6,
                fc1_W, fc1_b, fc2_W, fc2_b)
